# baseline (speedup 1.0000x reference)
import functools

import jax
import jax.numpy as jnp
from jax import lax
from jax.experimental import pallas as pl
from jax.experimental.pallas import tpu as pltpu
from jax.experimental.pallas import tpu_sc as plsc

N_NODES = 100000
D = 128
B = 10000
S = 32
FAN = S + 1

NC = 2
NS = 16
NW = NC * NS

R_C0 = 280
R_C1 = 360
BP = NS * (R_C0 + R_C1)
ROW_STRIDE = 40
NBUF = 2
KW = 14592

_INV_FAN = 1.0 / float(FAN)


def _fire(feat_hbm, idx1_v, idx2_v, row, b1, b2, sem):
    s = row * ROW_STRIDE
    pltpu.async_copy(feat_hbm.at[idx1_v.at[pl.ds(s, FAN)]], b1, sem)
    pltpu.async_copy(feat_hbm.at[idx2_v.at[pl.ds(s, FAN)]], b2, sem)


def _drain(feat_hbm, idx1_v, b1, b2, sem):
    pltpu.make_async_copy(feat_hbm.at[idx1_v.at[pl.ds(0, FAN)]], b1, sem).wait()
    pltpu.make_async_copy(feat_hbm.at[idx1_v.at[pl.ds(0, FAN)]], b2, sem).wait()


def _accum(buf, stage, k):
    inv = jnp.float32(_INV_FAN)
    for c in range(D // 16):
        d = pl.ds(16 * c, 16)
        s0 = buf[0, d]
        s1 = buf[1, d]
        s2 = buf[2, d]
        s3 = buf[3, d]
        for j in range(4, 32, 4):
            s0 = s0 + buf[j, d]
            s1 = s1 + buf[j + 1, d]
            s2 = s2 + buf[j + 2, d]
            s3 = s3 + buf[j + 3, d]
        s0 = s0 + buf[32, d]
        stage[k, d] = ((s0 + s1) + (s2 + s3)) * inv


def _sc_body(idx_hbm, perm_hbm, feat_hbm, out1_hbm, out2_hbm,
             idx1_v, idx2_v, buf1, buf2, st1, st2, *sems):
    ci = lax.axis_index("c")
    si = lax.axis_index("s")
    widx = ci * NS + si
    nrows = jnp.where(ci == 0, R_C0, R_C1)
    base = ci * (NS * R_C0) + si * nrows
    sem_g = sems[:NBUF]
    sem_w = sems[NBUF]
    sem_i = sems[NBUF + 1]

    pltpu.sync_copy(idx_hbm.at[pl.ds(widx * KW, KW)], idx1_v)

    K = 8

    def _p1_fire(r):
        s = r * ROW_STRIDE
        pltpu.async_copy(perm_hbm.at[idx1_v.at[pl.ds(s, FAN)]],
                         idx2_v.at[pl.ds(s, FAN)], sem_i)

    def _p1_wait():
        pltpu.make_async_copy(perm_hbm.at[pl.ds(0, FAN)],
                              idx2_v.at[pl.ds(0, FAN)], sem_i).wait()

    for r in range(K):
        _p1_fire(r)

    def p1_body(r, carry):
        _p1_fire(r + K)
        _p1_wait()
        return carry

    lax.fori_loop(0, nrows + NBUF - K, p1_body, 0)
    for _ in range(K):
        _p1_wait()

    for k in range(NBUF):
        _fire(feat_hbm, idx1_v, idx2_v, k, buf1.at[k], buf2.at[k], sem_g[k])

    def body(t, carry):
        for k in range(NBUF):
            r = NBUF * t + k

            @pl.when(t > 0)
            def _():
                pltpu.make_async_copy(
                    st1.at[k], out1_hbm.at[pl.ds(0, D)], sem_w).wait()
                pltpu.make_async_copy(
                    st2.at[k], out2_hbm.at[pl.ds(0, D)], sem_w).wait()

            _drain(feat_hbm, idx1_v, buf1.at[k], buf2.at[k], sem_g[k])
            _accum(buf1.at[k], st1, k)
            _accum(buf2.at[k], st2, k)
            gr = (base + r) * D
            pltpu.async_copy(st1.at[k], out1_hbm.at[pl.ds(gr, D)], sem_w)
            pltpu.async_copy(st2.at[k], out2_hbm.at[pl.ds(gr, D)], sem_w)
            _fire(feat_hbm, idx1_v, idx2_v, r + NBUF,
                  buf1.at[k], buf2.at[k], sem_g[k])
        return carry

    lax.fori_loop(0, nrows // NBUF, body, 0)

    for k in range(NBUF):
        _drain(feat_hbm, idx1_v, buf1.at[k], buf2.at[k], sem_g[k])
        pltpu.make_async_copy(st1.at[k], out1_hbm.at[pl.ds(0, D)], sem_w).wait()
        pltpu.make_async_copy(st2.at[k], out2_hbm.at[pl.ds(0, D)], sem_w).wait()


_sc_aggregate = functools.partial(
    pl.kernel,
    mesh=plsc.VectorSubcoreMesh(core_axis_name="c", subcore_axis_name="s"),
    out_type=[jax.ShapeDtypeStruct((BP * D,), jnp.float32),
              jax.ShapeDtypeStruct((BP * D,), jnp.float32)],
    scratch_types=[
        pltpu.VMEM((KW,), jnp.int32),
        pltpu.VMEM((KW,), jnp.int32),
        pltpu.VMEM((NBUF, FAN, D), jnp.float32),
        pltpu.VMEM((NBUF, FAN, D), jnp.float32),
        pltpu.VMEM((NBUF, D), jnp.float32),
        pltpu.VMEM((NBUF, D), jnp.float32),
    ] + [pltpu.SemaphoreType.DMA] * (NBUF + 2),
)(_sc_body)


TC_BLK = 512


def _tc_body(a1_ref, a2_ref, w_ref, alpha_ref, o1_ref, o2_ref):
    w = w_ref[...]
    al = alpha_ref[0, 0]
    dn = (((1,), (1,)), ((), ()))
    y1 = lax.dot_general(w, a1_ref[...], dn,
                         preferred_element_type=jnp.float32,
                         precision=lax.Precision.HIGHEST)
    o1_ref[...] = jnp.where(y1 >= 0, y1, al * y1)
    y2 = lax.dot_general(w, a2_ref[...], dn,
                         preferred_element_type=jnp.float32,
                         precision=lax.Precision.HIGHEST)
    o2_ref[...] = jnp.where(y2 >= 0, y2, al * y2)


def _tc_combine(agg1, agg2, W, alpha2d):
    return pl.pallas_call(
        _tc_body,
        grid=(BP // TC_BLK,),
        in_specs=[
            pl.BlockSpec((TC_BLK, D), lambda i: (i, 0)),
            pl.BlockSpec((TC_BLK, D), lambda i: (i, 0)),
            pl.BlockSpec((D, D), lambda i: (0, 0)),
            pl.BlockSpec(memory_space=pltpu.SMEM),
        ],
        out_specs=[
            pl.BlockSpec((D, TC_BLK), lambda i: (0, i)),
            pl.BlockSpec((D, TC_BLK), lambda i: (0, i)),
        ],
        out_shape=[jax.ShapeDtypeStruct((D, BP), jnp.float32),
                   jax.ShapeDtypeStruct((D, BP), jnp.float32)],
    )(agg1, agg2, W, alpha2d)


def kernel(nodes, neigh_idx, perm, features, W, alpha):
    idx1 = jnp.concatenate([neigh_idx, nodes[:, None]], axis=1)
    idxp = jnp.zeros((BP, ROW_STRIDE), jnp.int32).at[:B, :FAN].set(idx1)
    part_a = idxp[:NS * R_C0].reshape(NS, R_C0 * ROW_STRIDE)
    part_b = idxp[NS * R_C0:].reshape(NS, R_C1 * ROW_STRIDE)
    idx_hbm = (jnp.zeros((NW, KW), jnp.int32)
               .at[:NS, :R_C0 * ROW_STRIDE].set(part_a)
               .at[NS:, :R_C1 * ROW_STRIDE].set(part_b).reshape(NW * KW))

    agg1, agg2 = _sc_aggregate(idx_hbm, perm, features)
    out1, out2 = _tc_combine(agg1.reshape(BP, D), agg2.reshape(BP, D),
                             W, alpha.reshape(1, 1))
    return out1[:, :B], out2[:, :B]


# speedup vs baseline: 1.1414x; 1.1414x over previous
import functools

import jax
import jax.numpy as jnp
from jax import lax
from jax.experimental import pallas as pl
from jax.experimental.pallas import tpu as pltpu
from jax.experimental.pallas import tpu_sc as plsc

N_NODES = 100000
D = 128
B = 10000
S = 32
FAN = S + 1

NC = 2
NS = 16
NW = NC * NS

R_C0 = 360
R_C1 = 280
BP = NS * (R_C0 + R_C1)
ROW_STRIDE = 40
NBUF = 2
KW = 14592

_INV_FAN = 1.0 / float(FAN)


def _fire(feat_hbm, idx1_v, idx2_v, row, b1, b2, sem):
    s = row * ROW_STRIDE
    pltpu.async_copy(feat_hbm.at[idx1_v.at[pl.ds(s, FAN)]], b1, sem)
    pltpu.async_copy(feat_hbm.at[idx2_v.at[pl.ds(s, FAN)]], b2, sem)


def _drain(feat_hbm, idx1_v, b1, b2, sem):
    pltpu.make_async_copy(feat_hbm.at[idx1_v.at[pl.ds(0, FAN)]], b1, sem).wait()
    pltpu.make_async_copy(feat_hbm.at[idx1_v.at[pl.ds(0, FAN)]], b2, sem).wait()


def _accum(buf, stage, k):
    inv = jnp.float32(_INV_FAN)
    for c in range(D // 16):
        d = pl.ds(16 * c, 16)
        s0 = buf[0, d]
        s1 = buf[1, d]
        s2 = buf[2, d]
        s3 = buf[3, d]
        for j in range(4, 32, 4):
            s0 = s0 + buf[j, d]
            s1 = s1 + buf[j + 1, d]
            s2 = s2 + buf[j + 2, d]
            s3 = s3 + buf[j + 3, d]
        s0 = s0 + buf[32, d]
        stage[k, d] = ((s0 + s1) + (s2 + s3)) * inv


def _sc_body(idx_hbm, perm_hbm, feat_hbm, out1_hbm, out2_hbm,
             idx1_v, idx2_v, buf1, buf2, st1, st2, *sems):
    ci = lax.axis_index("c")
    si = lax.axis_index("s")
    widx = ci * NS + si
    nrows = jnp.where(ci == 0, R_C0, R_C1)
    base = ci * (NS * R_C0) + si * nrows
    sem_g = sems[:NBUF]
    sem_w = sems[NBUF]
    sem_i = sems[NBUF + 1]

    pltpu.sync_copy(idx_hbm.at[pl.ds(widx * KW, KW)], idx1_v)

    K = 8

    def _p1_fire(r):
        s = r * ROW_STRIDE
        pltpu.async_copy(perm_hbm.at[idx1_v.at[pl.ds(s, FAN)]],
                         idx2_v.at[pl.ds(s, FAN)], sem_i)

    def _p1_wait():
        pltpu.make_async_copy(perm_hbm.at[pl.ds(0, FAN)],
                              idx2_v.at[pl.ds(0, FAN)], sem_i).wait()

    for r in range(K):
        _p1_fire(r)

    def p1_body(r, carry):
        _p1_fire(r + K)
        _p1_wait()
        return carry

    lax.fori_loop(0, nrows + NBUF - K, p1_body, 0)
    for _ in range(K):
        _p1_wait()

    for k in range(NBUF):
        _fire(feat_hbm, idx1_v, idx2_v, k, buf1.at[k], buf2.at[k], sem_g[k])

    def body(t, carry):
        for k in range(NBUF):
            r = NBUF * t + k

            @pl.when(t > 0)
            def _():
                pltpu.make_async_copy(
                    st1.at[k], out1_hbm.at[pl.ds(0, D)], sem_w).wait()
                pltpu.make_async_copy(
                    st2.at[k], out2_hbm.at[pl.ds(0, D)], sem_w).wait()

            _drain(feat_hbm, idx1_v, buf1.at[k], buf2.at[k], sem_g[k])
            _accum(buf1.at[k], st1, k)
            _accum(buf2.at[k], st2, k)
            gr = (base + r) * D
            pltpu.async_copy(st1.at[k], out1_hbm.at[pl.ds(gr, D)], sem_w)
            pltpu.async_copy(st2.at[k], out2_hbm.at[pl.ds(gr, D)], sem_w)
            _fire(feat_hbm, idx1_v, idx2_v, r + NBUF,
                  buf1.at[k], buf2.at[k], sem_g[k])
        return carry

    lax.fori_loop(0, nrows // NBUF, body, 0)

    for k in range(NBUF):
        _drain(feat_hbm, idx1_v, buf1.at[k], buf2.at[k], sem_g[k])
        pltpu.make_async_copy(st1.at[k], out1_hbm.at[pl.ds(0, D)], sem_w).wait()
        pltpu.make_async_copy(st2.at[k], out2_hbm.at[pl.ds(0, D)], sem_w).wait()


_sc_aggregate = functools.partial(
    pl.kernel,
    mesh=plsc.VectorSubcoreMesh(core_axis_name="c", subcore_axis_name="s"),
    out_type=[jax.ShapeDtypeStruct((BP * D,), jnp.float32),
              jax.ShapeDtypeStruct((BP * D,), jnp.float32)],
    scratch_types=[
        pltpu.VMEM((KW,), jnp.int32),
        pltpu.VMEM((KW,), jnp.int32),
        pltpu.VMEM((NBUF, FAN, D), jnp.float32),
        pltpu.VMEM((NBUF, FAN, D), jnp.float32),
        pltpu.VMEM((NBUF, D), jnp.float32),
        pltpu.VMEM((NBUF, D), jnp.float32),
    ] + [pltpu.SemaphoreType.DMA] * (NBUF + 2),
)(_sc_body)


TC_BLK = 512


def _tc_body(a1_ref, a2_ref, w_ref, alpha_ref, o1_ref, o2_ref):
    w = w_ref[...]
    al = alpha_ref[0, 0]
    dn = (((1,), (1,)), ((), ()))
    y1 = lax.dot_general(w, a1_ref[...], dn,
                         preferred_element_type=jnp.float32,
                         precision=lax.Precision.HIGHEST)
    o1_ref[...] = jnp.where(y1 >= 0, y1, al * y1)
    y2 = lax.dot_general(w, a2_ref[...], dn,
                         preferred_element_type=jnp.float32,
                         precision=lax.Precision.HIGHEST)
    o2_ref[...] = jnp.where(y2 >= 0, y2, al * y2)


def _tc_combine(agg1, agg2, W, alpha2d):
    return pl.pallas_call(
        _tc_body,
        grid=(BP // TC_BLK,),
        in_specs=[
            pl.BlockSpec((TC_BLK, D), lambda i: (i, 0)),
            pl.BlockSpec((TC_BLK, D), lambda i: (i, 0)),
            pl.BlockSpec((D, D), lambda i: (0, 0)),
            pl.BlockSpec(memory_space=pltpu.SMEM),
        ],
        out_specs=[
            pl.BlockSpec((D, TC_BLK), lambda i: (0, i)),
            pl.BlockSpec((D, TC_BLK), lambda i: (0, i)),
        ],
        out_shape=[jax.ShapeDtypeStruct((D, BP), jnp.float32),
                   jax.ShapeDtypeStruct((D, BP), jnp.float32)],
    )(agg1, agg2, W, alpha2d)


def kernel(nodes, neigh_idx, perm, features, W, alpha):
    idx1 = jnp.concatenate([neigh_idx, nodes[:, None]], axis=1)
    idxp = jnp.zeros((BP, ROW_STRIDE), jnp.int32).at[:B, :FAN].set(idx1)
    part_a = idxp[:NS * R_C0].reshape(NS, R_C0 * ROW_STRIDE)
    part_b = idxp[NS * R_C0:].reshape(NS, R_C1 * ROW_STRIDE)
    idx_hbm = (jnp.zeros((NW, KW), jnp.int32)
               .at[:NS, :R_C0 * ROW_STRIDE].set(part_a)
               .at[NS:, :R_C1 * ROW_STRIDE].set(part_b).reshape(NW * KW))

    agg1, agg2 = _sc_aggregate(idx_hbm, perm, features)
    out1, out2 = _tc_combine(agg1.reshape(BP, D), agg2.reshape(BP, D),
                             W, alpha.reshape(1, 1))
    return out1[:, :B], out2[:, :B]


# speedup vs baseline: 1.1506x; 1.0081x over previous
import functools

import jax
import jax.numpy as jnp
from jax import lax
from jax.experimental import pallas as pl
from jax.experimental.pallas import tpu as pltpu
from jax.experimental.pallas import tpu_sc as plsc

N_NODES = 100000
D = 128
B = 10000
S = 32
FAN = S + 1

NC = 2
NS = 16
NW = NC * NS

R_C0 = 364
R_C1 = 276
BP = NS * (R_C0 + R_C1)
ROW_STRIDE = 40
NBUF = 2
KW = (R_C0 + 2 * NBUF) * ROW_STRIDE

_INV_FAN = 1.0 / float(FAN)


def _fire(feat_hbm, idx1_v, idx2_v, row, b1, b2, sem):
    s = row * ROW_STRIDE
    pltpu.async_copy(feat_hbm.at[idx1_v.at[pl.ds(s, FAN)]], b1, sem)
    pltpu.async_copy(feat_hbm.at[idx2_v.at[pl.ds(s, FAN)]], b2, sem)


def _drain(feat_hbm, idx1_v, b1, b2, sem):
    pltpu.make_async_copy(feat_hbm.at[idx1_v.at[pl.ds(0, FAN)]], b1, sem).wait()
    pltpu.make_async_copy(feat_hbm.at[idx1_v.at[pl.ds(0, FAN)]], b2, sem).wait()


def _accum(buf, stage, k):
    inv = jnp.float32(_INV_FAN)
    for c in range(D // 16):
        d = pl.ds(16 * c, 16)
        s0 = buf[0, d]
        s1 = buf[1, d]
        s2 = buf[2, d]
        s3 = buf[3, d]
        for j in range(4, 32, 4):
            s0 = s0 + buf[j, d]
            s1 = s1 + buf[j + 1, d]
            s2 = s2 + buf[j + 2, d]
            s3 = s3 + buf[j + 3, d]
        s0 = s0 + buf[32, d]
        stage[k, d] = ((s0 + s1) + (s2 + s3)) * inv


def _sc_body(idx_hbm, perm_hbm, feat_hbm, out1_hbm, out2_hbm,
             idx1_v, idx2_v, buf1, buf2, st1, st2, *sems):
    ci = lax.axis_index("c")
    si = lax.axis_index("s")
    widx = ci * NS + si
    nrows = jnp.where(ci == 0, R_C0, R_C1)
    base = ci * (NS * R_C0) + si * nrows
    sem_g = sems[:NBUF]
    sem_w = sems[NBUF]
    sem_i = sems[NBUF + 1]

    pltpu.sync_copy(idx_hbm.at[pl.ds(widx * KW, KW)], idx1_v)

    K = 8

    def _p1_fire(r):
        s = r * ROW_STRIDE
        pltpu.async_copy(perm_hbm.at[idx1_v.at[pl.ds(s, FAN)]],
                         idx2_v.at[pl.ds(s, FAN)], sem_i)

    def _p1_wait():
        pltpu.make_async_copy(perm_hbm.at[pl.ds(0, FAN)],
                              idx2_v.at[pl.ds(0, FAN)], sem_i).wait()

    for r in range(K):
        _p1_fire(r)

    def p1_body(r, carry):
        _p1_fire(r + K)
        _p1_wait()
        return carry

    lax.fori_loop(0, nrows + NBUF - K, p1_body, 0)
    for _ in range(K):
        _p1_wait()

    for k in range(NBUF):
        _fire(feat_hbm, idx1_v, idx2_v, k, buf1.at[k], buf2.at[k], sem_g[k])

    def body(t, carry):
        for k in range(NBUF):
            r = NBUF * t + k

            @pl.when(t > 0)
            def _():
                pltpu.make_async_copy(
                    st1.at[k], out1_hbm.at[pl.ds(0, D)], sem_w).wait()
                pltpu.make_async_copy(
                    st2.at[k], out2_hbm.at[pl.ds(0, D)], sem_w).wait()

            _drain(feat_hbm, idx1_v, buf1.at[k], buf2.at[k], sem_g[k])
            _accum(buf1.at[k], st1, k)
            _accum(buf2.at[k], st2, k)
            gr = (base + r) * D
            pltpu.async_copy(st1.at[k], out1_hbm.at[pl.ds(gr, D)], sem_w)
            pltpu.async_copy(st2.at[k], out2_hbm.at[pl.ds(gr, D)], sem_w)
            _fire(feat_hbm, idx1_v, idx2_v, r + NBUF,
                  buf1.at[k], buf2.at[k], sem_g[k])
        return carry

    lax.fori_loop(0, nrows // NBUF, body, 0)

    for k in range(NBUF):
        _drain(feat_hbm, idx1_v, buf1.at[k], buf2.at[k], sem_g[k])
        pltpu.make_async_copy(st1.at[k], out1_hbm.at[pl.ds(0, D)], sem_w).wait()
        pltpu.make_async_copy(st2.at[k], out2_hbm.at[pl.ds(0, D)], sem_w).wait()


_sc_aggregate = functools.partial(
    pl.kernel,
    mesh=plsc.VectorSubcoreMesh(core_axis_name="c", subcore_axis_name="s"),
    out_type=[jax.ShapeDtypeStruct((BP * D,), jnp.float32),
              jax.ShapeDtypeStruct((BP * D,), jnp.float32)],
    scratch_types=[
        pltpu.VMEM((KW,), jnp.int32),
        pltpu.VMEM((KW,), jnp.int32),
        pltpu.VMEM((NBUF, FAN, D), jnp.float32),
        pltpu.VMEM((NBUF, FAN, D), jnp.float32),
        pltpu.VMEM((NBUF, D), jnp.float32),
        pltpu.VMEM((NBUF, D), jnp.float32),
    ] + [pltpu.SemaphoreType.DMA] * (NBUF + 2),
)(_sc_body)


TC_BLK = 512


def _tc_body(a1_ref, a2_ref, w_ref, alpha_ref, o1_ref, o2_ref):
    w = w_ref[...]
    al = alpha_ref[0, 0]
    dn = (((1,), (1,)), ((), ()))
    y1 = lax.dot_general(w, a1_ref[...], dn,
                         preferred_element_type=jnp.float32,
                         precision=lax.Precision.HIGHEST)
    o1_ref[...] = jnp.where(y1 >= 0, y1, al * y1)
    y2 = lax.dot_general(w, a2_ref[...], dn,
                         preferred_element_type=jnp.float32,
                         precision=lax.Precision.HIGHEST)
    o2_ref[...] = jnp.where(y2 >= 0, y2, al * y2)


def _tc_combine(agg1, agg2, W, alpha2d):
    return pl.pallas_call(
        _tc_body,
        grid=(BP // TC_BLK,),
        in_specs=[
            pl.BlockSpec((TC_BLK, D), lambda i: (i, 0)),
            pl.BlockSpec((TC_BLK, D), lambda i: (i, 0)),
            pl.BlockSpec((D, D), lambda i: (0, 0)),
            pl.BlockSpec(memory_space=pltpu.SMEM),
        ],
        out_specs=[
            pl.BlockSpec((D, TC_BLK), lambda i: (0, i)),
            pl.BlockSpec((D, TC_BLK), lambda i: (0, i)),
        ],
        out_shape=[jax.ShapeDtypeStruct((D, BP), jnp.float32),
                   jax.ShapeDtypeStruct((D, BP), jnp.float32)],
    )(agg1, agg2, W, alpha2d)


def kernel(nodes, neigh_idx, perm, features, W, alpha):
    idx1 = jnp.concatenate([neigh_idx, nodes[:, None]], axis=1)
    idxp = jnp.zeros((BP, ROW_STRIDE), jnp.int32).at[:B, :FAN].set(idx1)
    part_a = idxp[:NS * R_C0].reshape(NS, R_C0 * ROW_STRIDE)
    part_b = idxp[NS * R_C0:].reshape(NS, R_C1 * ROW_STRIDE)
    idx_hbm = (jnp.zeros((NW, KW), jnp.int32)
               .at[:NS, :R_C0 * ROW_STRIDE].set(part_a)
               .at[NS:, :R_C1 * ROW_STRIDE].set(part_b).reshape(NW * KW))

    agg1, agg2 = _sc_aggregate(idx_hbm, perm, features)
    out1, out2 = _tc_combine(agg1.reshape(BP, D), agg2.reshape(BP, D),
                             W, alpha.reshape(1, 1))
    return out1[:, :B], out2[:, :B]


# speedup vs baseline: 1.1661x; 1.0135x over previous
import functools

import jax
import jax.numpy as jnp
from jax import lax
from jax.experimental import pallas as pl
from jax.experimental.pallas import tpu as pltpu
from jax.experimental.pallas import tpu_sc as plsc

N_NODES = 100000
D = 128
B = 10000
S = 32
FAN = S + 1

NC = 2
NS = 16
NW = NC * NS

R_C0 = 372
R_C1 = 268
BP = NS * (R_C0 + R_C1)
ROW_STRIDE = 40
NBUF = 2
KW = (R_C0 + 2 * NBUF) * ROW_STRIDE

_INV_FAN = 1.0 / float(FAN)


def _fire(feat_hbm, idx1_v, idx2_v, row, b1, b2, sem):
    s = row * ROW_STRIDE
    pltpu.async_copy(feat_hbm.at[idx1_v.at[pl.ds(s, FAN)]], b1, sem)
    pltpu.async_copy(feat_hbm.at[idx2_v.at[pl.ds(s, FAN)]], b2, sem)


def _drain(feat_hbm, idx1_v, b1, b2, sem):
    pltpu.make_async_copy(feat_hbm.at[idx1_v.at[pl.ds(0, FAN)]], b1, sem).wait()
    pltpu.make_async_copy(feat_hbm.at[idx1_v.at[pl.ds(0, FAN)]], b2, sem).wait()


def _accum(buf, stage, k):
    inv = jnp.float32(_INV_FAN)
    for c in range(D // 16):
        d = pl.ds(16 * c, 16)
        s0 = buf[0, d]
        s1 = buf[1, d]
        s2 = buf[2, d]
        s3 = buf[3, d]
        for j in range(4, 32, 4):
            s0 = s0 + buf[j, d]
            s1 = s1 + buf[j + 1, d]
            s2 = s2 + buf[j + 2, d]
            s3 = s3 + buf[j + 3, d]
        s0 = s0 + buf[32, d]
        stage[k, d] = ((s0 + s1) + (s2 + s3)) * inv


def _sc_body(idx_hbm, perm_hbm, feat_hbm, out1_hbm, out2_hbm,
             idx1_v, idx2_v, buf1, buf2, st1, st2, *sems):
    ci = lax.axis_index("c")
    si = lax.axis_index("s")
    widx = ci * NS + si
    nrows = jnp.where(ci == 0, R_C0, R_C1)
    base = ci * (NS * R_C0) + si * nrows
    sem_g = sems[:NBUF]
    sem_w = sems[NBUF]
    sem_i = sems[NBUF + 1]

    pltpu.sync_copy(idx_hbm.at[pl.ds(widx * KW, KW)], idx1_v)

    K = 8

    def _p1_fire(r):
        s = r * ROW_STRIDE
        pltpu.async_copy(perm_hbm.at[idx1_v.at[pl.ds(s, FAN)]],
                         idx2_v.at[pl.ds(s, FAN)], sem_i)

    def _p1_wait():
        pltpu.make_async_copy(perm_hbm.at[pl.ds(0, FAN)],
                              idx2_v.at[pl.ds(0, FAN)], sem_i).wait()

    for r in range(K):
        _p1_fire(r)

    def p1_body(r, carry):
        _p1_fire(r + K)
        _p1_wait()
        return carry

    lax.fori_loop(0, nrows + NBUF - K, p1_body, 0)
    for _ in range(K):
        _p1_wait()

    for k in range(NBUF):
        _fire(feat_hbm, idx1_v, idx2_v, k, buf1.at[k], buf2.at[k], sem_g[k])

    def body(t, carry):
        for k in range(NBUF):
            r = NBUF * t + k

            @pl.when(t > 0)
            def _():
                pltpu.make_async_copy(
                    st1.at[k], out1_hbm.at[pl.ds(0, D)], sem_w).wait()
                pltpu.make_async_copy(
                    st2.at[k], out2_hbm.at[pl.ds(0, D)], sem_w).wait()

            _drain(feat_hbm, idx1_v, buf1.at[k], buf2.at[k], sem_g[k])
            _accum(buf1.at[k], st1, k)
            _accum(buf2.at[k], st2, k)
            gr = (base + r) * D
            pltpu.async_copy(st1.at[k], out1_hbm.at[pl.ds(gr, D)], sem_w)
            pltpu.async_copy(st2.at[k], out2_hbm.at[pl.ds(gr, D)], sem_w)
            _fire(feat_hbm, idx1_v, idx2_v, r + NBUF,
                  buf1.at[k], buf2.at[k], sem_g[k])
        return carry

    lax.fori_loop(0, nrows // NBUF, body, 0)

    for k in range(NBUF):
        _drain(feat_hbm, idx1_v, buf1.at[k], buf2.at[k], sem_g[k])
        pltpu.make_async_copy(st1.at[k], out1_hbm.at[pl.ds(0, D)], sem_w).wait()
        pltpu.make_async_copy(st2.at[k], out2_hbm.at[pl.ds(0, D)], sem_w).wait()


_sc_aggregate = functools.partial(
    pl.kernel,
    mesh=plsc.VectorSubcoreMesh(core_axis_name="c", subcore_axis_name="s"),
    out_type=[jax.ShapeDtypeStruct((BP * D,), jnp.float32),
              jax.ShapeDtypeStruct((BP * D,), jnp.float32)],
    scratch_types=[
        pltpu.VMEM((KW,), jnp.int32),
        pltpu.VMEM((KW,), jnp.int32),
        pltpu.VMEM((NBUF, FAN, D), jnp.float32),
        pltpu.VMEM((NBUF, FAN, D), jnp.float32),
        pltpu.VMEM((NBUF, D), jnp.float32),
        pltpu.VMEM((NBUF, D), jnp.float32),
    ] + [pltpu.SemaphoreType.DMA] * (NBUF + 2),
)(_sc_body)


TC_BLK = 512


def _tc_body(a1_ref, a2_ref, w_ref, alpha_ref, o1_ref, o2_ref):
    w = w_ref[...]
    al = alpha_ref[0, 0]
    dn = (((1,), (1,)), ((), ()))
    y1 = lax.dot_general(w, a1_ref[...], dn,
                         preferred_element_type=jnp.float32,
                         precision=lax.Precision.HIGHEST)
    o1_ref[...] = jnp.where(y1 >= 0, y1, al * y1)
    y2 = lax.dot_general(w, a2_ref[...], dn,
                         preferred_element_type=jnp.float32,
                         precision=lax.Precision.HIGHEST)
    o2_ref[...] = jnp.where(y2 >= 0, y2, al * y2)


def _tc_combine(agg1, agg2, W, alpha2d):
    return pl.pallas_call(
        _tc_body,
        grid=(BP // TC_BLK,),
        in_specs=[
            pl.BlockSpec((TC_BLK, D), lambda i: (i, 0)),
            pl.BlockSpec((TC_BLK, D), lambda i: (i, 0)),
            pl.BlockSpec((D, D), lambda i: (0, 0)),
            pl.BlockSpec(memory_space=pltpu.SMEM),
        ],
        out_specs=[
            pl.BlockSpec((D, TC_BLK), lambda i: (0, i)),
            pl.BlockSpec((D, TC_BLK), lambda i: (0, i)),
        ],
        out_shape=[jax.ShapeDtypeStruct((D, BP), jnp.float32),
                   jax.ShapeDtypeStruct((D, BP), jnp.float32)],
    )(agg1, agg2, W, alpha2d)


def kernel(nodes, neigh_idx, perm, features, W, alpha):
    idx1 = jnp.concatenate([neigh_idx, nodes[:, None]], axis=1)
    idxp = jnp.zeros((BP, ROW_STRIDE), jnp.int32).at[:B, :FAN].set(idx1)
    part_a = idxp[:NS * R_C0].reshape(NS, R_C0 * ROW_STRIDE)
    part_b = idxp[NS * R_C0:].reshape(NS, R_C1 * ROW_STRIDE)
    idx_hbm = (jnp.zeros((NW, KW), jnp.int32)
               .at[:NS, :R_C0 * ROW_STRIDE].set(part_a)
               .at[NS:, :R_C1 * ROW_STRIDE].set(part_b).reshape(NW * KW))

    agg1, agg2 = _sc_aggregate(idx_hbm, perm, features)
    out1, out2 = _tc_combine(agg1.reshape(BP, D), agg2.reshape(BP, D),
                             W, alpha.reshape(1, 1))
    return out1[:, :B], out2[:, :B]
